# baseline (speedup 1.0000x reference)
import functools

import jax
import jax.numpy as jnp
from jax import lax
from jax.experimental import pallas as pl
from jax.experimental.pallas import tpu as pltpu
from jax.experimental.pallas import tpu_sc as plsc

_EMBED = 64
_SCALE = 8.0
_LANES = 16
_NUM_CORES = 2
_NUM_SUBCORES = 16
_NW = _NUM_CORES * _NUM_SUBCORES
_CHUNK = 128
_NBUF = 2


@functools.lru_cache(maxsize=None)
def _make_lookup(batch: int, seq: int, vocab: int):
    n_total = batch * seq
    assert batch % _CHUNK == 0
    n_chunks_all = n_total // _CHUNK
    assert n_chunks_all % _NW == 0
    per_w = n_chunks_all // _NW
    assert per_w % _NBUF == 0 and per_w >= 2 * _NBUF
    bblocks = batch // _CHUNK

    mesh = plsc.VectorSubcoreMesh(
        core_axis_name="c", subcore_axis_name="s", num_cores=_NUM_CORES
    )

    @functools.partial(
        pl.kernel,
        mesh=mesh,
        compiler_params=pltpu.CompilerParams(
            needs_layout_passes=False, use_tc_tiling_on_sc=True
        ),
        out_type=jax.ShapeDtypeStruct((seq, _EMBED, batch), jnp.float32),
        scratch_types=[
            pltpu.VMEM((per_w, _CHUNK), jnp.int32),
            *[pltpu.VMEM((_CHUNK,), jnp.int32) for _ in range(_NBUF)],
            *[pltpu.VMEM((_CHUNK,), jnp.int32) for _ in range(_NBUF)],
            *[pltpu.VMEM((_CHUNK, 2 * _EMBED), jnp.float32) for _ in range(_NBUF)],
            *[pltpu.VMEM((_EMBED, _CHUNK), jnp.float32) for _ in range(_NBUF)],
            *[pltpu.SemaphoreType.DMA for _ in range(_NBUF)],
            *[pltpu.SemaphoreType.DMA for _ in range(_NBUF)],
        ],
    )
    def lookup(idx_hbm, table_hbm, out_hbm, idx_v, *rest):
        gi = rest[:_NBUF]
        cb = rest[_NBUF : 2 * _NBUF]
        gbuf = rest[2 * _NBUF : 3 * _NBUF]
        obuf = rest[3 * _NBUF : 4 * _NBUF]
        gsems = rest[4 * _NBUF : 5 * _NBUF]
        ssems = rest[5 * _NBUF :]
        wid = lax.axis_index("s") * _NUM_CORES + lax.axis_index("c")
        cbase = wid * per_w

        pltpu.sync_copy(idx_hbm.at[wid], idx_v)

        def start_gather(b, t):
            for g in range(_CHUNK // _LANES):
                sl = pl.ds(g * _LANES, _LANES)
                xv = idx_v[t, sl]
                gi[b][sl] = lax.shift_right_logical(xv, 1)
                cb[b][sl] = lax.mul(lax.bitwise_and(xv, 1), _EMBED)
            pltpu.async_copy(
                table_hbm.at[gi[b].at[pl.ds(0, _CHUNK)]], gbuf[b], gsems[b]
            )

        def wait_gather(b):
            pltpu.make_async_copy(
                table_hbm.at[gi[b].at[pl.ds(0, _CHUNK)]], gbuf[b], gsems[b]
            ).wait()

        def select_scale_transpose(b):
            @pl.loop(0, _EMBED)
            def _(e):
                for g in range(_CHUNK // _LANES):
                    sl = pl.ds(g * _LANES, _LANES)
                    jv = lax.iota(jnp.int32, _LANES) + g * _LANES
                    cv = cb[b][sl] + e
                    val = plsc.load_gather(gbuf[b], [jv, cv])
                    obuf[b][e, sl] = val * _SCALE

        def dst(t):
            c = cbase + t
            s = c // bblocks
            b0 = c % bblocks
            return out_hbm.at[s, :, pl.ds(b0 * _CHUNK, _CHUNK)]

        def start_store(b, t):
            pltpu.async_copy(obuf[b], dst(t), ssems[b])

        def wait_store(b, t):
            pltpu.make_async_copy(obuf[b], dst(t), ssems[b]).wait()

        for b in range(_NBUF):
            start_gather(b, b)

        @pl.loop(0, per_w - _NBUF, step=_NBUF)
        def _(t0):
            for b in range(_NBUF):
                t = t0 + b
                wait_gather(b)
                select_scale_transpose(b)
                start_store(b, t)
                wait_store(b, t)
                start_gather(b, t + _NBUF)

        for b in range(_NBUF):
            t = per_w - _NBUF + b
            wait_gather(b)
            select_scale_transpose(b)
            start_store(b, t)
            wait_store(b, t)

    return lookup


def kernel(x, embedding):
    batch, seq = x.shape
    vocab, embed = embedding.shape
    idx = x.T.reshape(_NW, (batch * seq) // (_NW * _CHUNK), _CHUNK)
    idx = idx.astype(jnp.int32)
    table = embedding.reshape(vocab // 2, 2 * embed)
    out = _make_lookup(batch, seq, vocab)(idx, table)
    return out.transpose(2, 0, 1)


# speedup vs baseline: 1.0267x; 1.0267x over previous
import functools

import jax
import jax.numpy as jnp
from jax import lax
from jax.experimental import pallas as pl
from jax.experimental.pallas import tpu as pltpu
from jax.experimental.pallas import tpu_sc as plsc

_EMBED = 64
_SCALE = 8.0
_LANES = 16
_NUM_CORES = 2
_NUM_SUBCORES = 16
_NW = _NUM_CORES * _NUM_SUBCORES
_CHUNK = 128
_NBUF = 4


@functools.lru_cache(maxsize=None)
def _make_lookup(batch: int, seq: int, vocab: int):
    n_total = batch * seq
    assert batch % _CHUNK == 0
    n_chunks_all = n_total // _CHUNK
    assert n_chunks_all % _NW == 0
    per_w = n_chunks_all // _NW
    assert per_w % _NBUF == 0 and per_w >= 2 * _NBUF
    bblocks = batch // _CHUNK

    mesh = plsc.VectorSubcoreMesh(
        core_axis_name="c", subcore_axis_name="s", num_cores=_NUM_CORES
    )

    @functools.partial(
        pl.kernel,
        mesh=mesh,
        compiler_params=pltpu.CompilerParams(
            needs_layout_passes=False, use_tc_tiling_on_sc=True
        ),
        out_type=jax.ShapeDtypeStruct((seq, _EMBED, batch), jnp.float32),
        scratch_types=[
            pltpu.VMEM((per_w, _CHUNK), jnp.int32),
            *[pltpu.VMEM((_CHUNK,), jnp.int32) for _ in range(_NBUF)],
            *[pltpu.VMEM((_CHUNK,), jnp.int32) for _ in range(_NBUF)],
            *[pltpu.VMEM((_CHUNK, 2 * _EMBED), jnp.float32) for _ in range(_NBUF)],
            *[pltpu.VMEM((_EMBED, _CHUNK), jnp.float32) for _ in range(_NBUF)],
            *[pltpu.SemaphoreType.DMA for _ in range(_NBUF)],
            *[pltpu.SemaphoreType.DMA for _ in range(_NBUF)],
        ],
    )
    def lookup(idx_hbm, table_hbm, out_hbm, idx_v, *rest):
        gi = rest[:_NBUF]
        cb = rest[_NBUF : 2 * _NBUF]
        gbuf = rest[2 * _NBUF : 3 * _NBUF]
        obuf = rest[3 * _NBUF : 4 * _NBUF]
        gsems = rest[4 * _NBUF : 5 * _NBUF]
        ssems = rest[5 * _NBUF :]
        wid = lax.axis_index("s") * _NUM_CORES + lax.axis_index("c")
        cbase = wid * per_w

        pltpu.sync_copy(idx_hbm.at[wid], idx_v)

        def start_gather(b, t):
            for g in range(_CHUNK // _LANES):
                sl = pl.ds(g * _LANES, _LANES)
                xv = idx_v[t, sl]
                gi[b][sl] = lax.shift_right_logical(xv, 1)
                cb[b][sl] = lax.mul(lax.bitwise_and(xv, 1), _EMBED)
            pltpu.async_copy(
                table_hbm.at[gi[b].at[pl.ds(0, _CHUNK)]], gbuf[b], gsems[b]
            )

        def wait_gather(b):
            pltpu.make_async_copy(
                table_hbm.at[gi[b].at[pl.ds(0, _CHUNK)]], gbuf[b], gsems[b]
            ).wait()

        def select_scale_transpose(b):
            @pl.loop(0, _EMBED)
            def _(e):
                for g in range(_CHUNK // _LANES):
                    sl = pl.ds(g * _LANES, _LANES)
                    jv = lax.iota(jnp.int32, _LANES) + g * _LANES
                    cv = cb[b][sl] + e
                    val = plsc.load_gather(gbuf[b], [jv, cv])
                    obuf[b][e, sl] = val * _SCALE

        def dst(t):
            c = cbase + t
            s = c // bblocks
            b0 = c % bblocks
            return out_hbm.at[s, :, pl.ds(b0 * _CHUNK, _CHUNK)]

        def start_store(b, t):
            pltpu.async_copy(obuf[b], dst(t), ssems[b])

        def wait_store(b, t):
            pltpu.make_async_copy(obuf[b], dst(t), ssems[b]).wait()

        for b in range(_NBUF):
            start_gather(b, b)

        for b in range(_NBUF):
            wait_gather(b)
            select_scale_transpose(b)
            start_store(b, b)
            start_gather(b, b + _NBUF)

        @pl.loop(_NBUF, per_w - _NBUF, step=_NBUF)
        def _(t0):
            for b in range(_NBUF):
                t = t0 + b
                wait_gather(b)
                wait_store(b, t - _NBUF)
                select_scale_transpose(b)
                start_store(b, t)
                start_gather(b, t + _NBUF)

        for b in range(_NBUF):
            t = per_w - _NBUF + b
            wait_gather(b)
            wait_store(b, t - _NBUF)
            select_scale_transpose(b)
            start_store(b, t)
            wait_store(b, t)

    return lookup


def kernel(x, embedding):
    batch, seq = x.shape
    vocab, embed = embedding.shape
    idx = x.T.reshape(_NW, (batch * seq) // (_NW * _CHUNK), _CHUNK)
    idx = idx.astype(jnp.int32)
    table = embedding.reshape(vocab // 2, 2 * embed)
    out = _make_lookup(batch, seq, vocab)(idx, table)
    return out.transpose(2, 0, 1)


# speedup vs baseline: 2.1983x; 2.1412x over previous
import functools

import jax
import jax.numpy as jnp
from jax import lax
from jax.experimental import pallas as pl
from jax.experimental.pallas import tpu as pltpu
from jax.experimental.pallas import tpu_sc as plsc

_EMBED = 64
_SCALE = 8.0
_LANES = 16
_NUM_CORES = 2
_NUM_SUBCORES = 16
_NW = _NUM_CORES * _NUM_SUBCORES
_CHUNK = 128
_NBUF = 4


@functools.lru_cache(maxsize=None)
def _make_lookup(n_total: int):
    assert n_total % (_NW * _CHUNK) == 0
    per_w = n_total // _NW
    n_chunks = per_w // _CHUNK
    assert n_chunks >= 2 * _NBUF and n_chunks % _NBUF == 0

    mesh = plsc.VectorSubcoreMesh(
        core_axis_name="c", subcore_axis_name="s", num_cores=_NUM_CORES
    )

    @functools.partial(
        pl.kernel,
        mesh=mesh,
        compiler_params=pltpu.CompilerParams(
            needs_layout_passes=False, use_tc_tiling_on_sc=False
        ),
        out_type=jax.ShapeDtypeStruct((n_total, _EMBED), jnp.float32),
        scratch_types=[
            pltpu.VMEM((per_w,), jnp.int32),
            *[pltpu.VMEM((_CHUNK, _EMBED), jnp.float32) for _ in range(_NBUF)],
            *[pltpu.SemaphoreType.DMA for _ in range(_NBUF)],
            *[pltpu.SemaphoreType.DMA for _ in range(_NBUF)],
        ],
    )
    def lookup(idx_hbm, table_hbm, out_hbm, idx_v, *rest):
        bufs = rest[:_NBUF]
        gsems = rest[_NBUF : 2 * _NBUF]
        ssems = rest[2 * _NBUF :]
        wid = lax.axis_index("s") * _NUM_CORES + lax.axis_index("c")
        base = wid * per_w

        pltpu.sync_copy(idx_hbm.at[wid], idx_v)

        def start_gather(b, t):
            pltpu.async_copy(
                table_hbm.at[idx_v.at[pl.ds(t * _CHUNK, _CHUNK)]],
                bufs[b],
                gsems[b],
            )

        def wait_gather(b, t):
            pltpu.make_async_copy(
                table_hbm.at[idx_v.at[pl.ds(t * _CHUNK, _CHUNK)]],
                bufs[b],
                gsems[b],
            ).wait()

        def scale(b):
            buf = bufs[b]

            @pl.loop(0, _CHUNK // 4)
            def _(g):
                for r in range(4):
                    for c in range(_EMBED // _LANES):
                        sl = pl.ds(c * _LANES, _LANES)
                        buf[g * 4 + r, sl] = buf[g * 4 + r, sl] * _SCALE

        def start_store(b, t):
            pltpu.async_copy(
                bufs[b],
                out_hbm.at[pl.ds(base + t * _CHUNK, _CHUNK)],
                ssems[b],
            )

        def wait_store(b, t):
            pltpu.make_async_copy(
                bufs[b],
                out_hbm.at[pl.ds(base + t * _CHUNK, _CHUNK)],
                ssems[b],
            ).wait()

        for b in range(_NBUF):
            start_gather(b, b)

        @pl.loop(0, n_chunks - _NBUF, step=_NBUF)
        def _(cbase):
            for b in range(_NBUF):
                t = cbase + b
                wait_gather(b, t)
                scale(b)
                start_store(b, t)
                wait_store(b, t)
                start_gather(b, t + _NBUF)

        for b in range(_NBUF):
            t = n_chunks - _NBUF + b
            wait_gather(b, t)
            scale(b)
            start_store(b, t)
            wait_store(b, t)

    return lookup


def kernel(x, embedding):
    batch, seq = x.shape
    n_total = batch * seq
    idx = x.reshape(_NW, n_total // _NW).astype(jnp.int32)
    out = _make_lookup(n_total)(idx, embedding)
    return out.reshape(batch, seq, _EMBED)
